# baseline (speedup 1.0000x reference)
import jax
import jax.numpy as jnp
from jax.experimental import pallas as pl

_EPS = 1e-5


def _choose_block(n):
    for b in (8000, 5000, 4000, 2000, 1000):
        if n % b == 0:
            return b
    return n


def _stats_kernel(x_ref, s_ref, ss_ref):
    x = x_ref[...]

    @pl.when(pl.program_id(0) == 0)
    def _():
        s_ref[...] = jnp.zeros_like(s_ref)
        ss_ref[...] = jnp.zeros_like(ss_ref)

    s_ref[...] += jnp.sum(x, axis=0, keepdims=True)
    ss_ref[...] += jnp.sum(x * x, axis=0, keepdims=True)


def _affine_kernel(x_ref, sc_ref, sh_ref, o_ref):
    o_ref[...] = x_ref[...] * sc_ref[...] + sh_ref[...]


def _mm_stats_kernel(x_ref, w_ref, h_ref, s_ref, ss_ref):
    h = jnp.dot(x_ref[...], w_ref[...], preferred_element_type=jnp.float32)
    h_ref[...] = h

    @pl.when(pl.program_id(0) == 0)
    def _():
        s_ref[...] = jnp.zeros_like(s_ref)
        ss_ref[...] = jnp.zeros_like(ss_ref)

    s_ref[...] += jnp.sum(h, axis=0, keepdims=True)
    ss_ref[...] += jnp.sum(h * h, axis=0, keepdims=True)


def _affine_prelu_kernel(x_ref, sc_ref, sh_ref, a_ref, o_ref):
    h = x_ref[...] * sc_ref[...] + sh_ref[...]
    a = a_ref[0, 0]
    o_ref[...] = jnp.where(h > 0, h, a * h)


def _head_kernel(x_ref, w1_ref, b1_ref, w2_ref, b2_ref, o_ref):
    e1 = jnp.dot(x_ref[...], w1_ref[...],
                 preferred_element_type=jnp.float32) + b1_ref[...]
    e2 = jnp.dot(e1, w2_ref[...],
                 preferred_element_type=jnp.float32) + b2_ref[...]
    o_ref[...] = jnp.concatenate([e2, e1], axis=1)


def _pstats(x):
    n, f = x.shape
    b = _choose_block(n)
    nb = n // b
    s, ss = pl.pallas_call(
        _stats_kernel,
        grid=(nb,),
        in_specs=[pl.BlockSpec((b, f), lambda i: (i, 0))],
        out_specs=[pl.BlockSpec((1, f), lambda i: (0, 0)),
                   pl.BlockSpec((1, f), lambda i: (0, 0))],
        out_shape=[jax.ShapeDtypeStruct((1, f), jnp.float32),
                   jax.ShapeDtypeStruct((1, f), jnp.float32)],
    )(x)
    m = s[0] / n
    v = ss[0] / n - m * m
    return m, v


def _affine(x, sc, sh):
    n, f = x.shape
    b = _choose_block(n)
    nb = n // b
    return pl.pallas_call(
        _affine_kernel,
        grid=(nb,),
        in_specs=[pl.BlockSpec((b, f), lambda i: (i, 0)),
                  pl.BlockSpec((1, f), lambda i: (0, 0)),
                  pl.BlockSpec((1, f), lambda i: (0, 0))],
        out_specs=pl.BlockSpec((b, f), lambda i: (i, 0)),
        out_shape=jax.ShapeDtypeStruct((n, f), jnp.float32),
    )(x, sc.reshape(1, f), sh.reshape(1, f))


def _bn_pallas(x, g, bb):
    m, v = _pstats(x)
    sc = g / jnp.sqrt(v + _EPS)
    sh = bb - m * sc
    return _affine(x, sc, sh)


def _mm_stats(x, w):
    n, fin = x.shape
    fout = w.shape[1]
    b = _choose_block(n)
    nb = n // b
    h, s, ss = pl.pallas_call(
        _mm_stats_kernel,
        grid=(nb,),
        in_specs=[pl.BlockSpec((b, fin), lambda i: (i, 0)),
                  pl.BlockSpec((fin, fout), lambda i: (0, 0))],
        out_specs=[pl.BlockSpec((b, fout), lambda i: (i, 0)),
                   pl.BlockSpec((1, fout), lambda i: (0, 0)),
                   pl.BlockSpec((1, fout), lambda i: (0, 0))],
        out_shape=[jax.ShapeDtypeStruct((n, fout), jnp.float32),
                   jax.ShapeDtypeStruct((1, fout), jnp.float32),
                   jax.ShapeDtypeStruct((1, fout), jnp.float32)],
    )(x, w)
    m = s[0] / n
    v = ss[0] / n - m * m
    return h, m, v


def _affine_prelu(x, sc, sh, a):
    n, f = x.shape
    b = _choose_block(n)
    nb = n // b
    return pl.pallas_call(
        _affine_prelu_kernel,
        grid=(nb,),
        in_specs=[pl.BlockSpec((b, f), lambda i: (i, 0)),
                  pl.BlockSpec((1, f), lambda i: (0, 0)),
                  pl.BlockSpec((1, f), lambda i: (0, 0)),
                  pl.BlockSpec((1, 1), lambda i: (0, 0))],
        out_specs=pl.BlockSpec((b, f), lambda i: (i, 0)),
        out_shape=jax.ShapeDtypeStruct((n, f), jnp.float32),
    )(x, sc.reshape(1, f), sh.reshape(1, f), a.reshape(1, 1))


def _lnr_pallas(x, p):
    h, m, v = _mm_stats(x, p['W'].T)
    sc = p['g'] / jnp.sqrt(v + _EPS)
    sh = p['b'] - m * sc
    return _affine_prelu(h, sc, sh, p['a'])


def _head(ea, w1, b1, w2, b2):
    n = ea.shape[0]
    b = _choose_block(n)
    nb = n // b
    return pl.pallas_call(
        _head_kernel,
        grid=(nb,),
        in_specs=[pl.BlockSpec((b, ea.shape[1]), lambda i: (i, 0)),
                  pl.BlockSpec((w1.shape[0], 4), lambda i: (0, 0)),
                  pl.BlockSpec((1, 4), lambda i: (0, 0)),
                  pl.BlockSpec((4, 1), lambda i: (0, 0)),
                  pl.BlockSpec((1, 1), lambda i: (0, 0))],
        out_specs=pl.BlockSpec((b, 5), lambda i: (i, 0)),
        out_shape=jax.ShapeDtypeStruct((n, 5), jnp.float32),
    )(ea, w1, b1.reshape(1, 4), w2, b2.reshape(1, 1))


def _scatter_mean(data, ids, n):
    s = jax.ops.segment_sum(data, ids, num_segments=n)
    c = jax.ops.segment_sum(jnp.ones((data.shape[0],), data.dtype), ids,
                            num_segments=n)
    return s / jnp.clip(c, 1.0)[:, None]


def kernel(x, edge_index, edge_attr, u, batch, params):
    p = params
    row = edge_index[0]
    col = edge_index[1]
    n = x.shape[0]
    g = u.shape[0]

    x = jnp.concatenate([p['nodes_emb'][x[:, 0].astype(jnp.int32)],
                         x[:, 1:]], axis=1)
    edge_attr = jnp.concatenate(
        [p['edges_emb'][edge_attr[:, 0].astype(jnp.int32)],
         edge_attr[:, 1:]], axis=1)
    x = _bn_pallas(x, p['x_g'], p['x_b'])
    edge_attr = _bn_pallas(edge_attr, p['e_g'], p['e_b'])
    u = _bn_pallas(u, p['u_g'], p['u_b'])

    ub_row = None
    for lp in p['layers']:
        src = x[row]
        dst = x[col]
        if ub_row is None:
            ub_row = u[batch[row]]
        ea = jnp.concatenate([(src + dst) * 0.5, edge_attr, ub_row], axis=1)
        edge_attr = _lnr_pallas(ea, lp['edge'])
        if 'node1' in lp:
            h = jnp.concatenate([dst, edge_attr], axis=1)
            h = _lnr_pallas(h, lp['node1'])
            h = _scatter_mean(h, row, n)
            h = jnp.concatenate([h, u[batch]], axis=1)
            x = _lnr_pallas(h, lp['node2'])
        if 'glob' in lp:
            gu = jnp.concatenate([u, _scatter_mean(x, batch, g)], axis=1)
            u = _lnr_pallas(gu, lp['glob'])
            ub_row = None

    return _head(edge_attr, p['o1_W'].T, p['o1_b'], p['o2_W'].T, p['o2_b'])
